# baseline (speedup 1.0000x reference)
import functools

import jax
import jax.numpy as jnp
from jax.experimental import pallas as pl
from jax.experimental.pallas import tpu as pltpu


def _mm_body(adj_hbm, w_ref, out_ref, buf, sem, *, bm, nbuf, nblocks):
    half = bm // 2

    def cp(i, slot, h):
        return pltpu.make_async_copy(
            adj_hbm.at[pl.ds(i * bm + h * half, half), :],
            buf.at[slot, pl.ds(h * half, half)],
            sem.at[slot, h],
        )

    def start(i, slot):
        cp(i, slot, 0).start()
        cp(i, slot, 1).start()

    for s in range(nbuf - 1):
        start(s, s)

    def step(i, carry):
        nxt = i + (nbuf - 1)

        @pl.when(nxt < nblocks)
        def _():
            start(nxt, jax.lax.rem(nxt, nbuf))

        slot = jax.lax.rem(i, nbuf)
        cp(i, slot, 0).wait()
        cp(i, slot, 1).wait()
        out_ref[pl.ds(i * bm, bm), :] = jnp.dot(
            buf[slot], w_ref[...], preferred_element_type=jnp.float32
        )
        return carry

    jax.lax.fori_loop(0, nblocks, step, 0)


def kernel(adj, weight):
    m, k = adj.shape
    k2, n = weight.shape
    assert k == k2
    bm = 128
    nbuf = 4
    nblocks = m // bm
    return pl.pallas_call(
        functools.partial(_mm_body, bm=bm, nbuf=nbuf, nblocks=nblocks),
        in_specs=[
            pl.BlockSpec(memory_space=pltpu.HBM),
            pl.BlockSpec((k2, n), lambda: (0, 0)),
        ],
        out_specs=pl.BlockSpec((m, n), lambda: (0, 0)),
        out_shape=jax.ShapeDtypeStruct((m, n), jnp.float32),
        scratch_shapes=[
            pltpu.VMEM((nbuf, bm, k), jnp.float32),
            pltpu.SemaphoreType.DMA((nbuf, 2)),
        ],
    )(adj, weight)


# speedup vs baseline: 1.0151x; 1.0151x over previous
import jax
import jax.numpy as jnp
from jax.experimental import pallas as pl
from jax.experimental.pallas import tpu as pltpu


def _mm_body(adj_ref, w_ref, out_ref):
    out_ref[...] = jnp.dot(
        adj_ref[...], w_ref[...], preferred_element_type=jnp.float32
    )


def kernel(adj, weight):
    m, k = adj.shape
    k2, n = weight.shape
    assert k == k2
    bm = 256
    grid = (m // bm,)
    return pl.pallas_call(
        _mm_body,
        grid=grid,
        in_specs=[
            pl.BlockSpec((bm, k), lambda i: (i, 0)),
            pl.BlockSpec((k2, n), lambda i: (0, 0)),
        ],
        out_specs=pl.BlockSpec((bm, n), lambda i: (i, 0)),
        out_shape=jax.ShapeDtypeStruct((m, n), jnp.float32),
        compiler_params=pltpu.CompilerParams(
            dimension_semantics=("arbitrary",),
        ),
    )(adj, weight)
